# baseline (speedup 1.0000x reference)
import functools
import jax
import jax.numpy as jnp
import numpy as np
from jax.experimental import pallas as pl
from jax.experimental.pallas import tpu as pltpu

C = 128; NHEAD = 8; H = 128; S = 12; N = 10000; FUSED = C + 2 * H
DH = C // NHEAD
AVG_LOG = float(np.log(17.0))
NEG = -1e30


def _ln(x, g, b, eps=1e-5):
    m = jnp.mean(x, axis=-1, keepdims=True)
    v = jnp.mean((x - m) ** 2, axis=-1, keepdims=True)
    return (x - m) / jnp.sqrt(v + eps) * g + b


def _lrelu(x):
    return jnp.where(x >= 0, x, 0.01 * x)


def _tf_kernel(x_ref, wqkv_ref, bqkv_ref, wo_ref, bo_ref, ln1g_ref, ln1b_ref,
               w1_ref, b1_ref, w2_ref, b2_ref, ln2g_ref, ln2b_ref,
               tng_ref, tnb_ref, o_ref, kcat_ref, vcat_ref, *, rows):
    T = rows * S
    x = x_ref[...].reshape(T, C)
    qkv = x @ wqkv_ref[...].T + bqkv_ref[...]
    q = qkv[:, :C]
    k = qkv[:, C:2 * C]
    v = qkv[:, 2 * C:]
    lane = jax.lax.broadcasted_iota(jnp.int32, (1, C), 1) // DH
    for h in range(NHEAD):
        m = (lane == h).astype(jnp.float32)
        kcat_ref[h * T:(h + 1) * T, :] = k * m
        vcat_ref[h * T:(h + 1) * T, :] = v * m
    logits = jax.lax.dot_general(q, kcat_ref[...], (((1,), (1,)), ((), ())))
    ti = jax.lax.broadcasted_iota(jnp.int32, (T, NHEAD * T), 0) // S
    tj = (jax.lax.broadcasted_iota(jnp.int32, (T, NHEAD * T), 1) % T) // S
    logits = jnp.where(ti == tj, logits * (1.0 / np.sqrt(float(DH))), NEG)
    l3 = logits.reshape(T, NHEAD, T)
    l3 = l3 - jnp.max(l3, axis=-1, keepdims=True)
    e = jnp.exp(l3)
    a = (e / jnp.sum(e, axis=-1, keepdims=True)).reshape(T, NHEAD * T)
    o = jax.lax.dot_general(a, vcat_ref[...], (((1,), (0,)), ((), ())))
    o = o @ wo_ref[...].T + bo_ref[...]
    x = _ln(x + o, ln1g_ref[...], ln1b_ref[...])
    f = jnp.maximum(x @ w1_ref[...].T + b1_ref[...], 0.0) @ w2_ref[...].T + b2_ref[...]
    t = _ln(x + f, ln2g_ref[...], ln2b_ref[...])
    t = _ln(t, tng_ref[...], tnb_ref[...])
    o_ref[...] = t.reshape(rows, S, C)


def _transformer(x_tab, p, tng, tnb, rows=8):
    B = x_tab.shape[0]
    grid = B // rows
    w = lambda shape: pl.BlockSpec(shape, lambda i: (0,) * len(shape))
    return pl.pallas_call(
        functools.partial(_tf_kernel, rows=rows),
        grid=(grid,),
        in_specs=[pl.BlockSpec((rows, S, C), lambda i: (i, 0, 0)),
                  w((3 * C, C)), w((3 * C,)), w((C, C)), w((C,)),
                  w((C,)), w((C,)), w((C, C)), w((C,)), w((C, C)), w((C,)),
                  w((C,)), w((C,)), w((C,)), w((C,))],
        out_specs=pl.BlockSpec((rows, S, C), lambda i: (i, 0, 0)),
        out_shape=jax.ShapeDtypeStruct((B, S, C), jnp.float32),
        scratch_shapes=[pltpu.VMEM((NHEAD * rows * S, C), jnp.float32),
                        pltpu.VMEM((NHEAD * rows * S, C), jnp.float32)],
    )(x_tab, p['Wqkv'], p['bqkv'], p['Wo'], p['bo'], p['ln1_g'], p['ln1_b'],
      p['W1'], p['b1'], p['W2'], p['b2'], p['ln2_g'], p['ln2_b'], tng, tnb)


def _y12_kernel(x_ref, w12_ref, o_ref):
    o_ref[...] = x_ref[...] @ w12_ref[...]


def _node_pre(x_pad, w12, blk=512):
    n = x_pad.shape[0]
    return pl.pallas_call(
        _y12_kernel,
        grid=(n // blk,),
        in_specs=[pl.BlockSpec((blk, C), lambda i: (i, 0)),
                  pl.BlockSpec((C, 2 * C), lambda i: (0, 0))],
        out_specs=pl.BlockSpec((blk, 2 * C), lambda i: (i, 0)),
        out_shape=jax.ShapeDtypeStruct((n, 2 * C), jnp.float32),
    )(x_pad, w12)


def _z_kernel(ea_ref, wc_ref, bc_ref, o_ref):
    o_ref[...] = ea_ref[...] @ wc_ref[...] + bc_ref[...]


def _edge_pre(edge_attr, wc, bc, blk=640):
    e = edge_attr.shape[0]
    return pl.pallas_call(
        _z_kernel,
        grid=(e // blk,),
        in_specs=[pl.BlockSpec((blk, H), lambda i: (i, 0)),
                  pl.BlockSpec((H, H), lambda i: (0, 0)),
                  pl.BlockSpec((H,), lambda i: (0,))],
        out_specs=pl.BlockSpec((blk, H), lambda i: (i, 0)),
        out_shape=jax.ShapeDtypeStruct((e, H), jnp.float32),
    )(edge_attr, wc, bc)


def _wc_kernel(we_ref, be_ref, wp3_ref, bpre_ref, wc_ref, bc_ref):
    wc_ref[...] = we_ref[...].T @ wp3_ref[...]
    bc_ref[...] = (be_ref[...].reshape(1, H) @ wp3_ref[...]).reshape(H) + bpre_ref[...]


def _combine_wc(we, be, wp3, bpre):
    w = lambda shape: pl.BlockSpec(shape, lambda: (0,) * len(shape))
    return pl.pallas_call(
        _wc_kernel,
        in_specs=[w((H, H)), w((H,)), w((H, H)), w((H,))],
        out_specs=[w((H, H)), w((H,))],
        out_shape=[jax.ShapeDtypeStruct((H, H), jnp.float32),
                   jax.ShapeDtypeStruct((H,), jnp.float32)],
    )(we, be, wp3, bpre)


def _post_kernel(x_ref, cnt_ref, sum_ref, sq_ref, mx_ref, mn_ref,
                 wx_ref, w1_ref, w2_ref, w3_ref, bpost_ref,
                 wlin_ref, blin_ref, bng_ref, bnb_ref, o_ref):
    x = x_ref[...]
    cnt = cnt_ref[...].reshape(-1, 1)
    cntc = jnp.maximum(cnt, 1.0)
    mean = sum_ref[...] / cntc
    msq = sq_ref[...] / cntc
    pos = cnt > 0
    mx = jnp.where(pos, mx_ref[...], 0.0)
    mn = jnp.where(pos, mn_ref[...], 0.0)
    std = jnp.sqrt(jnp.maximum(msq - mean * mean, 0.0) + 1e-5)
    agg = jnp.concatenate([mean, mx, mn, std], axis=-1)
    s_amp = jnp.log(cntc + 1.0) / AVG_LOG
    out = (x @ wx_ref[...] + agg @ w1_ref[...] + (agg * s_amp) @ w2_ref[...]
           + (agg / s_amp) @ w3_ref[...] + bpost_ref[...])
    out = out @ wlin_ref[...].T + blin_ref[...]
    out = out * (1.0 / np.sqrt(1.0 + 1e-5)) * bng_ref[...] + bnb_ref[...]
    o_ref[...] = (x + jnp.maximum(out, 0.0)) * 0.5


def _node_post(x_pad, cnt_pad, ssum, ssq, smx, smn, pp, bng, bnb, blk=512):
    n = x_pad.shape[0]
    wpost_t = pp['Wpost'].T
    wx, w1, w2, w3 = (wpost_t[:H], wpost_t[H:5 * H], wpost_t[5 * H:9 * H],
                      wpost_t[9 * H:])
    w = lambda shape: pl.BlockSpec(shape, lambda i: (0,) * len(shape))
    r = lambda width: pl.BlockSpec((blk, width), lambda i: (i, 0))
    return pl.pallas_call(
        _post_kernel,
        grid=(n // blk,),
        in_specs=[r(H), pl.BlockSpec((blk,), lambda i: (i,)),
                  r(H), r(H), r(H), r(H),
                  w((H, H)), w((4 * H, H)), w((4 * H, H)), w((4 * H, H)),
                  w((H,)), w((H, H)), w((H,)), w((H,)), w((H,))],
        out_specs=r(H),
        out_shape=jax.ShapeDtypeStruct((n, H), jnp.float32),
    )(x_pad, cnt_pad, ssum, ssq, smx, smn, wx, w1, w2, w3, pp['bpost'],
      pp['Wlin'], pp['blin'], bng, bnb)


def _fuse_kernel(cls_ref, gs_ref, gd_ref, lng_ref, lnb_ref, w1_ref, b1_ref,
                 w2_ref, b2_ref, w3_ref, b3_ref, ng_ref, nb_ref, o_ref):
    x = jnp.concatenate([cls_ref[...], gs_ref[...], gd_ref[...]], axis=-1)
    hh = _ln(x, lng_ref[...], lnb_ref[...])
    hh = _lrelu(hh @ w1_ref[...].T + b1_ref[...])
    hh = _lrelu(hh @ w2_ref[...].T + b2_ref[...])
    hh = hh @ w3_ref[...].T + b3_ref[...]
    o_ref[...] = (x + _ln(hh, ng_ref[...], nb_ref[...])) * 0.5


def _fuse_mlp(cls, gs, gd, fp, blk=512):
    B = cls.shape[0]
    w = lambda shape: pl.BlockSpec(shape, lambda i: (0,) * len(shape))
    wspec = w((FUSED, FUSED)); vspec = w((FUSED,))
    return pl.pallas_call(
        _fuse_kernel,
        grid=(B // blk,),
        in_specs=[pl.BlockSpec((blk, C), lambda i: (i, 0)),
                  pl.BlockSpec((blk, H), lambda i: (i, 0)),
                  pl.BlockSpec((blk, H), lambda i: (i, 0)),
                  vspec, vspec, wspec, vspec, wspec, vspec, wspec, vspec,
                  vspec, vspec],
        out_specs=pl.BlockSpec((blk, FUSED), lambda i: (i, 0)),
        out_shape=jax.ShapeDtypeStruct((B, FUSED), jnp.float32),
    )(cls, gs, gd, fp['ln_g'], fp['ln_b'], fp['W1'], fp['b1'], fp['W2'],
      fp['b2'], fp['W3'], fp['b3'], fp['norm_g'], fp['norm_b'])


def kernel(x_tab, x_gnn, edge_index, edge_attr, params):
    p = params['pna']
    t = _transformer(x_tab, params['tab'], params['tab_norm_g'], params['tab_norm_b'])
    cls, rest = t[:, 0, :], t[:, 1:, :]

    wpre_t = p['Wpre'].T
    wc, bc = _combine_wc(p['We'], p['be'], wpre_t[2 * H:], p['bpre'])
    x_pad = jnp.pad(x_gnn, ((0, 240), (0, 0)))
    y12 = _node_pre(x_pad, jnp.concatenate([wpre_t[:H], wpre_t[H:2 * H]], axis=1))
    y1, y2 = y12[:N, :H], y12[:N, H:]
    z = _edge_pre(edge_attr, wc, bc)

    src, dst = edge_index[0], edge_index[1]
    h = y1[dst] + y2[src] + z
    cnt = jax.ops.segment_sum(jnp.ones((h.shape[0],), h.dtype), dst, num_segments=N)
    ssum = jax.ops.segment_sum(h, dst, num_segments=N)
    ssq = jax.ops.segment_sum(h * h, dst, num_segments=N)
    smx = jax.ops.segment_max(h, dst, num_segments=N)
    smn = jax.ops.segment_min(h, dst, num_segments=N)

    pad = lambda a: jnp.pad(a, ((0, 240),) + ((0, 0),) * (a.ndim - 1))
    g = _node_post(x_pad, pad(cnt), pad(ssum), pad(ssq), pad(smx), pad(smn),
                   p, params['bn_g'], params['bn_b'])[:N]

    b = cls.shape[0]
    src_b, dst_b = edge_index[0][:b], edge_index[1][:b]
    x = _fuse_mlp(cls, g[src_b], g[dst_b], params['fuse'])
    x_tab_out = jnp.concatenate([x[:, :C][:, None, :], rest], axis=1)
    g = g.at[src_b].set(x[:, C:C + H])
    g = g.at[dst_b].set(x[:, C + H:])
    return (x_tab_out, g, edge_attr)


# speedup vs baseline: 1.1268x; 1.1268x over previous
import functools
import jax
import jax.numpy as jnp
import numpy as np
from jax import lax
from jax.experimental import pallas as pl
from jax.experimental.pallas import tpu as pltpu
from jax.experimental.pallas import tpu_sc as plsc

C = 128; NHEAD = 8; H = 128; S = 12; N = 10000; FUSED = C + 2 * H
DH = C // NHEAD
AVG_LOG = float(np.log(17.0))
NEG = -1e30

NW = 32
ROUNDS = 3
NR = 112
NPAD = NW * ROUNDS * NR
CE = 3200
GG = 64
E_TOT = 160000


def _ln(x, g, b, eps=1e-5):
    m = jnp.mean(x, axis=-1, keepdims=True)
    v = jnp.mean((x - m) ** 2, axis=-1, keepdims=True)
    return (x - m) / jnp.sqrt(v + eps) * g + b


def _lrelu(x):
    return jnp.where(x >= 0, x, 0.01 * x)


def _tf_kernel(x_ref, wqkv_ref, bqkv_ref, wo_ref, bo_ref, ln1g_ref, ln1b_ref,
               w1_ref, b1_ref, w2_ref, b2_ref, ln2g_ref, ln2b_ref,
               tng_ref, tnb_ref, o_ref, kcat_ref, vcat_ref, *, rows):
    T = rows * S
    x = x_ref[...].reshape(T, C)
    qkv = x @ wqkv_ref[...].T + bqkv_ref[...]
    q = qkv[:, :C]
    k = qkv[:, C:2 * C]
    v = qkv[:, 2 * C:]
    lane = jax.lax.broadcasted_iota(jnp.int32, (1, C), 1) // DH
    for h in range(NHEAD):
        m = (lane == h).astype(jnp.float32)
        kcat_ref[h * T:(h + 1) * T, :] = k * m
        vcat_ref[h * T:(h + 1) * T, :] = v * m
    logits = jax.lax.dot_general(q, kcat_ref[...], (((1,), (1,)), ((), ())))
    ti = jax.lax.broadcasted_iota(jnp.int32, (T, NHEAD * T), 0) // S
    tj = (jax.lax.broadcasted_iota(jnp.int32, (T, NHEAD * T), 1) % T) // S
    logits = jnp.where(ti == tj, logits * (1.0 / np.sqrt(float(DH))), NEG)
    l3 = logits.reshape(T, NHEAD, T)
    l3 = l3 - jnp.max(l3, axis=-1, keepdims=True)
    e = jnp.exp(l3)
    a = (e / jnp.sum(e, axis=-1, keepdims=True)).reshape(T, NHEAD * T)
    o = jax.lax.dot_general(a, vcat_ref[...], (((1,), (0,)), ((), ())))
    o = o @ wo_ref[...].T + bo_ref[...]
    x = _ln(x + o, ln1g_ref[...], ln1b_ref[...])
    f = jnp.maximum(x @ w1_ref[...].T + b1_ref[...], 0.0) @ w2_ref[...].T + b2_ref[...]
    t = _ln(x + f, ln2g_ref[...], ln2b_ref[...])
    t = _ln(t, tng_ref[...], tnb_ref[...])
    o_ref[...] = t.reshape(rows, S, C)


def _transformer(x_tab, p, tng, tnb, rows=16):
    B = x_tab.shape[0]
    grid = B // rows
    w = lambda shape: pl.BlockSpec(shape, lambda i: (0,) * len(shape))
    return pl.pallas_call(
        functools.partial(_tf_kernel, rows=rows),
        grid=(grid,),
        in_specs=[pl.BlockSpec((rows, S, C), lambda i: (i, 0, 0)),
                  w((3 * C, C)), w((3 * C,)), w((C, C)), w((C,)),
                  w((C,)), w((C,)), w((C, C)), w((C,)), w((C, C)), w((C,)),
                  w((C,)), w((C,)), w((C,)), w((C,))],
        out_specs=pl.BlockSpec((rows, S, C), lambda i: (i, 0, 0)),
        out_shape=jax.ShapeDtypeStruct((B, S, C), jnp.float32),
        scratch_shapes=[pltpu.VMEM((NHEAD * rows * S, C), jnp.float32),
                        pltpu.VMEM((NHEAD * rows * S, C), jnp.float32)],
    )(x_tab, p['Wqkv'], p['bqkv'], p['Wo'], p['bo'], p['ln1_g'], p['ln1_b'],
      p['W1'], p['b1'], p['W2'], p['b2'], p['ln2_g'], p['ln2_b'], tng, tnb)


def _y12_kernel(x_ref, w12_ref, o_ref):
    o_ref[...] = x_ref[...] @ w12_ref[...]


def _node_pre(x_pad, w12, blk=512):
    n = x_pad.shape[0]
    return pl.pallas_call(
        _y12_kernel,
        grid=(n // blk,),
        in_specs=[pl.BlockSpec((blk, C), lambda i: (i, 0)),
                  pl.BlockSpec((C, 2 * C), lambda i: (0, 0))],
        out_specs=pl.BlockSpec((blk, 2 * C), lambda i: (i, 0)),
        out_shape=jax.ShapeDtypeStruct((n, 2 * C), jnp.float32),
    )(x_pad, w12)


def _z_kernel(ea_ref, wc_ref, bc_ref, o_ref):
    o_ref[...] = ea_ref[...] @ wc_ref[...] + bc_ref[...]


def _edge_pre(edge_attr, wc, bc, blk=640):
    e = edge_attr.shape[0]
    return pl.pallas_call(
        _z_kernel,
        grid=(e // blk,),
        in_specs=[pl.BlockSpec((blk, H), lambda i: (i, 0)),
                  pl.BlockSpec((H, H), lambda i: (0, 0)),
                  pl.BlockSpec((H,), lambda i: (0,))],
        out_specs=pl.BlockSpec((blk, H), lambda i: (i, 0)),
        out_shape=jax.ShapeDtypeStruct((e, H), jnp.float32),
    )(edge_attr, wc, bc)


def _wc_kernel(we_ref, be_ref, wp3_ref, bpre_ref, wc_ref, bc_ref):
    wc_ref[...] = we_ref[...].T @ wp3_ref[...]
    bc_ref[...] = (be_ref[...].reshape(1, H) @ wp3_ref[...]).reshape(H) + bpre_ref[...]


def _combine_wc(we, be, wp3, bpre):
    w = lambda shape: pl.BlockSpec(shape, lambda: (0,) * len(shape))
    return pl.pallas_call(
        _wc_kernel,
        in_specs=[w((H, H)), w((H,)), w((H, H)), w((H,))],
        out_specs=[w((H, H)), w((H,))],
        out_shape=[jax.ShapeDtypeStruct((H, H), jnp.float32),
                   jax.ShapeDtypeStruct((H,), jnp.float32)],
    )(we, be, wp3, bpre)


def _seg_kernel(y1_hbm, y2_hbm, z_hbm, dst_hbm, src_hbm,
                cnt_hbm, sum_hbm, sq_hbm, mx_hbm, mn_hbm,
                acc_s, acc_q, acc_mx, acc_mn, acc_c, y1b,
                dstb, srcb, locb, srcc, eidc, y2b, zb, sem, sem2):
    wid = lax.axis_index("s") * 2 + lax.axis_index("c")
    iota = lax.iota(jnp.int32, 16)
    lane0 = iota == 0
    ones = jnp.ones((16,), jnp.float32)

    def zero_idx_bufs(i, _):
        z16 = jnp.zeros((16,), jnp.int32)
        srcc[pl.ds(i * 16, 16)] = z16
        eidc[pl.ds(i * 16, 16)] = z16
        locb[pl.ds(i * 16, 16)] = z16
        return 0
    lax.fori_loop(0, CE // 16, zero_idx_bufs, 0)

    def one_round(r, _):
        base = (wid * ROUNDS + r) * NR

        def init_acc(i, _):
            row = i // 8
            col = (i % 8) * 16
            acc_s[row, pl.ds(col, 16)] = jnp.zeros((16,), jnp.float32)
            acc_q[row, pl.ds(col, 16)] = jnp.zeros((16,), jnp.float32)
            acc_mx[row, pl.ds(col, 16)] = jnp.full((16,), -3e38, jnp.float32)
            acc_mn[row, pl.ds(col, 16)] = jnp.full((16,), 3e38, jnp.float32)
            return 0
        lax.fori_loop(0, NR * 8, init_acc, 0)

        def init_cnt(i, _):
            acc_c[pl.ds(i * 16, 16)] = jnp.zeros((16,), jnp.float32)
            return 0
        lax.fori_loop(0, NR // 16, init_cnt, 0)

        pltpu.sync_copy(y1_hbm.at[pl.ds(base, NR)], y1b)

        def one_chunk(ch, _):
            c0 = ch * CE
            pltpu.sync_copy(dst_hbm.at[pl.ds(c0, CE)], dstb)
            pltpu.sync_copy(src_hbm.at[pl.ds(c0, CE)], srcb)

            def scan_step(i, off):
                d = dstb[pl.ds(i * 16, 16)]
                m = (d >= base) & (d < base + NR)
                nsel_v = jnp.sum(m.astype(jnp.int32))
                plsc.store_compressed(locb.at[pl.ds(off, 16)], d - base, mask=m)
                plsc.store_compressed(srcc.at[pl.ds(off, 16)],
                                      srcb[pl.ds(i * 16, 16)], mask=m)
                plsc.store_compressed(eidc.at[pl.ds(off, 16)],
                                      c0 + i * 16 + iota, mask=m)
                return off + nsel_v
            nsel = lax.fori_loop(0, CE // 16, scan_step, jnp.int32(0))

            def one_group(g, _):
                g0 = g * GG
                cp1 = pltpu.async_copy(y2_hbm.at[srcc.at[pl.ds(g0, GG)]], y2b, sem)
                cp2 = pltpu.async_copy(z_hbm.at[eidc.at[pl.ds(g0, GG)]], zb, sem2)
                cp1.wait()
                cp2.wait()

                def one_edge(j, _):
                    jv = jnp.full((16,), j, jnp.int32)
                    loc = plsc.load_gather(locb, [g0 + jv])
                    cnt_cur = plsc.load_gather(acc_c, [loc])
                    plsc.store_scatter(acc_c, [loc], cnt_cur + ones, mask=lane0)
                    for f in range(8):
                        fcol = f * 16 + iota
                        y1v = plsc.load_gather(y1b, [loc, fcol])
                        y2v = plsc.load_gather(y2b, [jv, fcol])
                        zv = plsc.load_gather(zb, [jv, fcol])
                        h = y1v + y2v + zv
                        plsc.addupdate_scatter(acc_s, [loc, fcol], h)
                        plsc.addupdate_scatter(acc_q, [loc, fcol], h * h)
                        mxc = plsc.load_gather(acc_mx, [loc, fcol])
                        plsc.store_scatter(acc_mx, [loc, fcol], jnp.maximum(mxc, h))
                        mnc = plsc.load_gather(acc_mn, [loc, fcol])
                        plsc.store_scatter(acc_mn, [loc, fcol], jnp.minimum(mnc, h))
                    return 0
                lax.fori_loop(0, jnp.minimum(nsel - g0, GG), one_edge, 0)
                return 0
            lax.fori_loop(0, (nsel + GG - 1) // GG, one_group, 0)
            return 0
        lax.fori_loop(0, E_TOT // CE, one_chunk, 0)

        pltpu.sync_copy(acc_s, sum_hbm.at[pl.ds(base, NR)])
        pltpu.sync_copy(acc_q, sq_hbm.at[pl.ds(base, NR)])
        pltpu.sync_copy(acc_mx, mx_hbm.at[pl.ds(base, NR)])
        pltpu.sync_copy(acc_mn, mn_hbm.at[pl.ds(base, NR)])
        pltpu.sync_copy(acc_c, cnt_hbm.at[pl.ds(base, NR)])
        return 0
    lax.fori_loop(0, ROUNDS, one_round, 0)


def _segment_aggregates(y1_pad, y2_pad, z, dst_arr, src_arr):
    f32 = jnp.float32
    k = pl.kernel(
        _seg_kernel,
        mesh=plsc.VectorSubcoreMesh(core_axis_name="c", subcore_axis_name="s"),
        compiler_params=pltpu.CompilerParams(needs_layout_passes=False),
        out_type=[jax.ShapeDtypeStruct((NPAD,), f32),
                  jax.ShapeDtypeStruct((NPAD, H), f32),
                  jax.ShapeDtypeStruct((NPAD, H), f32),
                  jax.ShapeDtypeStruct((NPAD, H), f32),
                  jax.ShapeDtypeStruct((NPAD, H), f32)],
        scratch_types=[
            pltpu.VMEM((NR, H), f32), pltpu.VMEM((NR, H), f32),
            pltpu.VMEM((NR, H), f32), pltpu.VMEM((NR, H), f32),
            pltpu.VMEM((NR,), f32), pltpu.VMEM((NR, H), f32),
            pltpu.VMEM((CE,), jnp.int32), pltpu.VMEM((CE,), jnp.int32),
            pltpu.VMEM((CE,), jnp.int32), pltpu.VMEM((CE,), jnp.int32),
            pltpu.VMEM((CE,), jnp.int32),
            pltpu.VMEM((GG, H), f32), pltpu.VMEM((GG, H), f32),
            pltpu.SemaphoreType.DMA, pltpu.SemaphoreType.DMA,
        ],
    )
    return k(y1_pad, y2_pad, z, dst_arr, src_arr)


def _post_kernel(x_ref, cnt_ref, sum_ref, sq_ref, mx_ref, mn_ref,
                 wx_ref, w1_ref, w2_ref, w3_ref, bpost_ref,
                 wlin_ref, blin_ref, bng_ref, bnb_ref, o_ref):
    x = x_ref[...]
    cnt = cnt_ref[...].reshape(-1, 1)
    cntc = jnp.maximum(cnt, 1.0)
    mean = sum_ref[...] / cntc
    msq = sq_ref[...] / cntc
    pos = cnt > 0
    mx = jnp.where(pos, mx_ref[...], 0.0)
    mn = jnp.where(pos, mn_ref[...], 0.0)
    std = jnp.sqrt(jnp.maximum(msq - mean * mean, 0.0) + 1e-5)
    agg = jnp.concatenate([mean, mx, mn, std], axis=-1)
    s_amp = jnp.log(cntc + 1.0) / AVG_LOG
    out = (x @ wx_ref[...] + agg @ w1_ref[...] + (agg * s_amp) @ w2_ref[...]
           + (agg / s_amp) @ w3_ref[...] + bpost_ref[...])
    out = out @ wlin_ref[...].T + blin_ref[...]
    out = out * (1.0 / np.sqrt(1.0 + 1e-5)) * bng_ref[...] + bnb_ref[...]
    o_ref[...] = (x + jnp.maximum(out, 0.0)) * 0.5


def _node_post(x_pad, cnt_pad, ssum, ssq, smx, smn, pp, bng, bnb, blk=512):
    n = x_pad.shape[0]
    wpost_t = pp['Wpost'].T
    wx, w1, w2, w3 = (wpost_t[:H], wpost_t[H:5 * H], wpost_t[5 * H:9 * H],
                      wpost_t[9 * H:])
    w = lambda shape: pl.BlockSpec(shape, lambda i: (0,) * len(shape))
    r = lambda width: pl.BlockSpec((blk, width), lambda i: (i, 0))
    return pl.pallas_call(
        _post_kernel,
        grid=(n // blk,),
        in_specs=[r(H), pl.BlockSpec((blk,), lambda i: (i,)),
                  r(H), r(H), r(H), r(H),
                  w((H, H)), w((4 * H, H)), w((4 * H, H)), w((4 * H, H)),
                  w((H,)), w((H, H)), w((H,)), w((H,)), w((H,))],
        out_specs=r(H),
        out_shape=jax.ShapeDtypeStruct((n, H), jnp.float32),
    )(x_pad, cnt_pad, ssum, ssq, smx, smn, wx, w1, w2, w3, pp['bpost'],
      pp['Wlin'], pp['blin'], bng, bnb)


def _fuse_kernel(cls_ref, gs_ref, gd_ref, lng_ref, lnb_ref, w1_ref, b1_ref,
                 w2_ref, b2_ref, w3_ref, b3_ref, ng_ref, nb_ref, o_ref):
    x = jnp.concatenate([cls_ref[...], gs_ref[...], gd_ref[...]], axis=-1)
    hh = _ln(x, lng_ref[...], lnb_ref[...])
    hh = _lrelu(hh @ w1_ref[...].T + b1_ref[...])
    hh = _lrelu(hh @ w2_ref[...].T + b2_ref[...])
    hh = hh @ w3_ref[...].T + b3_ref[...]
    o_ref[...] = (x + _ln(hh, ng_ref[...], nb_ref[...])) * 0.5


def _fuse_mlp(cls, gs, gd, fp, blk=512):
    B = cls.shape[0]
    w = lambda shape: pl.BlockSpec(shape, lambda i: (0,) * len(shape))
    wspec = w((FUSED, FUSED)); vspec = w((FUSED,))
    return pl.pallas_call(
        _fuse_kernel,
        grid=(B // blk,),
        in_specs=[pl.BlockSpec((blk, C), lambda i: (i, 0)),
                  pl.BlockSpec((blk, H), lambda i: (i, 0)),
                  pl.BlockSpec((blk, H), lambda i: (i, 0)),
                  vspec, vspec, wspec, vspec, wspec, vspec, wspec, vspec,
                  vspec, vspec],
        out_specs=pl.BlockSpec((blk, FUSED), lambda i: (i, 0)),
        out_shape=jax.ShapeDtypeStruct((B, FUSED), jnp.float32),
    )(cls, gs, gd, fp['ln_g'], fp['ln_b'], fp['W1'], fp['b1'], fp['W2'],
      fp['b2'], fp['W3'], fp['b3'], fp['norm_g'], fp['norm_b'])


def kernel(x_tab, x_gnn, edge_index, edge_attr, params):
    p = params['pna']
    t = _transformer(x_tab, params['tab'], params['tab_norm_g'], params['tab_norm_b'])
    cls, rest = t[:, 0, :], t[:, 1:, :]

    wpre_t = p['Wpre'].T
    wc, bc = _combine_wc(p['We'], p['be'], wpre_t[2 * H:], p['bpre'])
    x_pad = jnp.pad(x_gnn, ((0, NPAD - N), (0, 0)))
    y12 = _node_pre(x_pad, jnp.concatenate([wpre_t[:H], wpre_t[H:2 * H]], axis=1))
    y1_pad, y2_pad = y12[:, :H], y12[:, H:]
    z = _edge_pre(edge_attr, wc, bc)

    cnt, ssum, ssq, smx, smn = _segment_aggregates(
        y1_pad, y2_pad, z, edge_index[1], edge_index[0])

    g = _node_post(x_pad, cnt, ssum, ssq, smx, smn,
                   p, params['bn_g'], params['bn_b'])[:N]

    b = cls.shape[0]
    src_b, dst_b = edge_index[0][:b], edge_index[1][:b]
    x = _fuse_mlp(cls, g[src_b], g[dst_b], params['fuse'])
    x_tab_out = jnp.concatenate([x[:, :C][:, None, :], rest], axis=1)
    g = g.at[src_b].set(x[:, C:C + H])
    g = g.at[dst_b].set(x[:, C + H:])
    return (x_tab_out, g, edge_attr)
